# baseline (speedup 1.0000x reference)
import functools
import math

import jax
import jax.numpy as jnp
from jax import lax
from jax.experimental import pallas as pl
from jax.experimental.pallas import tpu as pltpu
from jax.experimental.pallas import tpu_sc as plsc

_SIZE = 32000
_SMOOTHING = 0.1
_CONF = 1.0 - _SMOOTHING
_EPS = _SMOOTHING / (_SIZE - 2)
_PAD = 0
_C_ROW = (_SIZE - 2) * _EPS * math.log(_EPS) + _CONF * math.log(_CONF)

_BR = 256
_BC = 6400

_N = 4096
_NC = 2
_NS = 16
_NW = _NC * _NS
_LANES = 16
_ROWS_PER_W = _N // _NW
_VPW = _ROWS_PER_W // _LANES


def _rowsum_tile(t_ref, x_ref, out_ref):
    r = pl.program_id(0)
    c = pl.program_id(1)

    @pl.when((r == 0) & (c == 0))
    def _init():
        out_ref[0, 0] = 0.0

    x = x_ref[...]
    t = t_ref[0]
    nonpad = t != _PAD
    rs = jnp.sum(x, axis=1, keepdims=True)
    partial = -_EPS * jnp.sum(jnp.where(nonpad, rs, 0.0))
    extra = _EPS * jnp.sum(jnp.where(nonpad, x[:, 0:1], 0.0)) \
        + _C_ROW * jnp.sum(nonpad.astype(jnp.float32))
    out_ref[0, 0] += partial + jnp.where(c == 0, extra, 0.0)


@functools.partial(
    pl.kernel,
    mesh=plsc.VectorSubcoreMesh(core_axis_name="c", subcore_axis_name="s"),
    out_type=jax.ShapeDtypeStruct((_NW, _LANES), jnp.float32),
    scratch_types=[
        pltpu.VMEM((_ROWS_PER_W,), jnp.int32),
        pltpu.VMEM((_ROWS_PER_W,), jnp.int32),
        pltpu.VMEM((_ROWS_PER_W,), jnp.float32),
        pltpu.VMEM((_LANES,), jnp.float32),
        pltpu.SemaphoreType.DMA,
    ],
)
def _sc_gather(xflat_hbm, t_hbm, out_hbm, t_v, idx_v, vals_v, acc_v, sem):
    wid = lax.axis_index("s") * _NC + lax.axis_index("c")
    base = wid * _ROWS_PER_W
    pltpu.sync_copy(t_hbm.at[pl.ds(base, _ROWS_PER_W)], t_v)
    for j in range(_VPW):
        t16 = t_v[pl.ds(j * _LANES, _LANES)]
        rows = lax.iota(jnp.int32, _LANES) + (base + j * _LANES)
        idx_v[pl.ds(j * _LANES, _LANES)] = rows * _SIZE + t16
    pltpu.async_copy(xflat_hbm.at[idx_v], vals_v, sem).wait()
    acc = jnp.zeros((_LANES,), jnp.float32)
    for j in range(_VPW):
        t16 = t_v[pl.ds(j * _LANES, _LANES)]
        v16 = vals_v[pl.ds(j * _LANES, _LANES)]
        acc = acc + jnp.where(t16 == _PAD, 0.0, v16)
    acc_v[...] = acc
    pltpu.sync_copy(acc_v, out_hbm.at[wid])


def kernel(x, target):
    N, V = x.shape
    assert V == _SIZE and N == _N and N % _BR == 0 and V % _BC == 0
    nr, nc = N // _BR, V // _BC
    t32 = target.astype(jnp.int32)
    t3 = t32.reshape(nr, _BR, 1)
    dense = pl.pallas_call(
        _rowsum_tile,
        grid=(nr, nc),
        in_specs=[
            pl.BlockSpec((1, _BR, 1), lambda r, c: (r, 0, 0)),
            pl.BlockSpec((_BR, _BC), lambda r, c: (r, c)),
        ],
        out_specs=pl.BlockSpec(
            (1, 1), lambda r, c: (0, 0), memory_space=pltpu.SMEM
        ),
        out_shape=jax.ShapeDtypeStruct((1, 1), jnp.float32),
    )(t3, x)
    gathered = _sc_gather(x.reshape(-1), t32)
    return dense[0, 0] - (_CONF - _EPS) * jnp.sum(gathered)


# speedup vs baseline: 3.0010x; 3.0010x over previous
import math

import jax
import jax.numpy as jnp
from jax.experimental import pallas as pl
from jax.experimental.pallas import tpu as pltpu

_SIZE = 32000
_SMOOTHING = 0.1
_CONF = 1.0 - _SMOOTHING
_EPS = _SMOOTHING / (_SIZE - 2)
_PAD = 0
_C_ROW = (_SIZE - 2) * _EPS * math.log(_EPS) + _CONF * math.log(_CONF)

_BR = 256
_BC = 6400


def _loss_tile(t_ref, x_ref, out_ref):
    r = pl.program_id(0)
    c = pl.program_id(1)

    @pl.when((r == 0) & (c == 0))
    def _init():
        out_ref[0, 0] = 0.0

    x = x_ref[...]
    t = t_ref[0]
    nonpad = t != _PAD
    col = jax.lax.broadcasted_iota(jnp.int32, (_BR, _BC), 1) + c * _BC
    rs = jnp.sum(x, axis=1, keepdims=True)
    g = jnp.sum(jnp.where(col == t, x, 0.0), axis=1, keepdims=True)
    per_row = -_EPS * rs - (_CONF - _EPS) * g
    per_row += jnp.where(c == 0, _EPS * x[:, 0:1] + _C_ROW, 0.0)
    out_ref[0, 0] += jnp.sum(jnp.where(nonpad, per_row, 0.0))


def kernel(x, target):
    N, V = x.shape
    assert V == _SIZE and N % _BR == 0 and V % _BC == 0
    nr, nc = N // _BR, V // _BC
    t3 = target.astype(jnp.int32).reshape(nr, _BR, 1)
    out = pl.pallas_call(
        _loss_tile,
        grid=(nr, nc),
        in_specs=[
            pl.BlockSpec((1, _BR, 1), lambda r, c: (r, 0, 0)),
            pl.BlockSpec((_BR, _BC), lambda r, c: (r, c)),
        ],
        out_specs=pl.BlockSpec(
            (1, 1), lambda r, c: (0, 0), memory_space=pltpu.SMEM
        ),
        out_shape=jax.ShapeDtypeStruct((1, 1), jnp.float32),
    )(t3, x)
    return out[0, 0]


# speedup vs baseline: 3.3231x; 1.1073x over previous
import math

import jax
import jax.numpy as jnp
from jax.experimental import pallas as pl
from jax.experimental.pallas import tpu as pltpu

_SIZE = 32000
_SMOOTHING = 0.1
_CONF = 1.0 - _SMOOTHING
_EPS = _SMOOTHING / (_SIZE - 2)
_PAD = 0
_C_ROW = (_SIZE - 2) * _EPS * math.log(_EPS) + _CONF * math.log(_CONF)

_BR = 128
_BC = _SIZE


def _loss_tile(t_ref, x_ref, out_ref):
    r = pl.program_id(0)

    @pl.when(r == 0)
    def _init():
        out_ref[0, 0] = 0.0

    x = x_ref[...]
    t = t_ref[0]
    nonpad = t != _PAD
    col = jax.lax.broadcasted_iota(jnp.int32, (_BR, _BC), 1)
    rs = jnp.sum(x, axis=1, keepdims=True)
    g = jnp.sum(jnp.where(col == t, x, 0.0), axis=1, keepdims=True)
    per_row = -_EPS * rs - (_CONF - _EPS) * g + _EPS * x[:, 0:1] + _C_ROW
    out_ref[0, 0] += jnp.sum(jnp.where(nonpad, per_row, 0.0))


def kernel(x, target):
    N, V = x.shape
    assert V == _SIZE and N % _BR == 0
    nr = N // _BR
    t3 = target.astype(jnp.int32).reshape(nr, _BR, 1)
    out = pl.pallas_call(
        _loss_tile,
        grid=(nr,),
        in_specs=[
            pl.BlockSpec((1, _BR, 1), lambda r: (r, 0, 0)),
            pl.BlockSpec((_BR, _BC), lambda r: (r, 0)),
        ],
        out_specs=pl.BlockSpec(
            (1, 1), lambda r: (0, 0), memory_space=pltpu.SMEM
        ),
        out_shape=jax.ShapeDtypeStruct((1, 1), jnp.float32),
    )(t3, x)
    return out[0, 0]
